# baseline (device time: 796280 ns/iter reference)
import jax
import jax.numpy as jnp
from jax import lax
from jax.experimental import pallas as pl
from jax.experimental.pallas import tpu as pltpu

N_DEV = 32


def _gelu(y):
    c = 0.7978845608028654
    return 0.5 * y * (1.0 + jnp.tanh(c * (y + 0.044715 * y * y * y)))


def kernel(x, w_mat):
    m_glob, _ = x.shape
    _, n = w_mat.shape
    m_per = m_glob // N_DEV

    def body(x_ref, w_ref, out_ref, send_buf, recv_buf, send_sems, recv_sems,
             credit_sem):
        my = lax.axis_index("i")
        left = lax.rem(my - 1 + N_DEV, N_DEV)
        right = lax.rem(my + 1, N_DEV)

        barrier = pltpu.get_barrier_semaphore()
        for nbr in (left, right):
            pl.semaphore_signal(barrier, inc=1, device_id=(nbr,),
                                device_id_type=pl.DeviceIdType.MESH)
        pl.semaphore_wait(barrier, 2)

        w = w_ref[:, :].astype(jnp.bfloat16)

        def chunk_partial(c):
            xs = x_ref[pl.ds(c * m_per, m_per), :].astype(jnp.bfloat16)
            return jnp.dot(xs, w, preferred_element_type=jnp.float32)

        def recv_desc(slot):
            return pltpu.make_async_remote_copy(
                src_ref=send_buf.at[slot], dst_ref=recv_buf.at[slot],
                send_sem=send_sems.at[slot], recv_sem=recv_sems.at[slot],
                device_id=(left,), device_id_type=pl.DeviceIdType.MESH)

        sends = {}
        for s in range(N_DEV - 1):
            slot = s % 2
            c = lax.rem(my - 1 - s + 2 * N_DEV, N_DEV)
            part = chunk_partial(c)
            if s == 0:
                wire = part.astype(jnp.bfloat16)
            else:
                pslot = (s - 1) % 2
                recv_desc(pslot).wait_recv()
                wire = (part + recv_buf[pslot, :, :].astype(jnp.float32)
                        ).astype(jnp.bfloat16)
            if s >= 2:
                sends[s - 2].wait_send()
                pl.semaphore_wait(credit_sem, 1)
            send_buf[slot, :, :] = wire
            rdma = pltpu.make_async_remote_copy(
                src_ref=send_buf.at[slot], dst_ref=recv_buf.at[slot],
                send_sem=send_sems.at[slot], recv_sem=recv_sems.at[slot],
                device_id=(right,), device_id_type=pl.DeviceIdType.MESH)
            rdma.start()
            sends[s] = rdma
            if s >= 1:
                pl.semaphore_signal(credit_sem, inc=1, device_id=(left,),
                                    device_id_type=pl.DeviceIdType.MESH)

        last_slot = (N_DEV - 2) % 2
        recv_desc(last_slot).wait_recv()
        part = chunk_partial(my)
        out_ref[:, :] = _gelu(part + recv_buf[last_slot, :, :].astype(jnp.float32))
        pl.semaphore_signal(credit_sem, inc=1, device_id=(left,),
                            device_id_type=pl.DeviceIdType.MESH)
        sends[N_DEV - 3].wait_send()
        sends[N_DEV - 2].wait_send()
        pl.semaphore_wait(credit_sem, 2)

    return pl.pallas_call(
        body,
        out_shape=jax.ShapeDtypeStruct((m_per, n), jnp.float32),
        in_specs=[pl.BlockSpec(memory_space=pltpu.VMEM),
                  pl.BlockSpec(memory_space=pltpu.VMEM)],
        out_specs=pl.BlockSpec(memory_space=pltpu.VMEM),
        scratch_shapes=[
            pltpu.VMEM((2, m_per, n), jnp.bfloat16),
            pltpu.VMEM((2, m_per, n), jnp.bfloat16),
            pltpu.SemaphoreType.DMA((2,)),
            pltpu.SemaphoreType.DMA((2,)),
            pltpu.SemaphoreType.REGULAR,
        ],
        compiler_params=pltpu.CompilerParams(collective_id=0),
    )(x, w_mat)


# device time: 733862 ns/iter; 1.0851x vs baseline; 1.0851x over previous
import jax
import jax.numpy as jnp
from jax import lax
from jax.experimental import pallas as pl
from jax.experimental.pallas import tpu as pltpu

N_DEV = 32


def _gelu(y):
    c = 0.7978845608028654
    return 0.5 * y * (1.0 + jnp.tanh(c * (y + 0.044715 * y * y * y)))


class _Ring:

    def __init__(self, *, send_to, recv_from, w_half, send_buf, recv_buf,
                 send_sems, recv_sems, credit_sem):
        self.send_to = send_to
        self.recv_from = recv_from
        self.w_half = w_half
        self.send_buf = send_buf
        self.recv_buf = recv_buf
        self.send_sems = send_sems
        self.recv_sems = recv_sems
        self.credit_sem = credit_sem
        self.sends = {}

    def recv_desc(self, slot):
        return pltpu.make_async_remote_copy(
            src_ref=self.send_buf.at[slot], dst_ref=self.recv_buf.at[slot],
            send_sem=self.send_sems.at[slot], recv_sem=self.recv_sems.at[slot],
            device_id=(self.recv_from,), device_id_type=pl.DeviceIdType.MESH)

    def step(self, s, part):
        slot = s % 2
        if s == 0:
            wire = part.astype(jnp.bfloat16)
        else:
            pslot = (s - 1) % 2
            self.recv_desc(pslot).wait_recv()
            wire = (part + self.recv_buf[pslot, :, :].astype(jnp.float32)
                    ).astype(jnp.bfloat16)
        if s >= 2:
            self.sends[s - 2].wait_send()
            pl.semaphore_wait(self.credit_sem, 1)
        self.send_buf[slot, :, :] = wire
        rdma = pltpu.make_async_remote_copy(
            src_ref=self.send_buf.at[slot], dst_ref=self.recv_buf.at[slot],
            send_sem=self.send_sems.at[slot], recv_sem=self.recv_sems.at[slot],
            device_id=(self.send_to,), device_id_type=pl.DeviceIdType.MESH)
        rdma.start()
        self.sends[s] = rdma
        if s >= 1:
            pl.semaphore_signal(self.credit_sem, inc=1,
                                device_id=(self.recv_from,),
                                device_id_type=pl.DeviceIdType.MESH)

    def finish(self, part):
        last_slot = (N_DEV - 2) % 2
        self.recv_desc(last_slot).wait_recv()
        out = _gelu(part + self.recv_buf[last_slot, :, :].astype(jnp.float32))
        pl.semaphore_signal(self.credit_sem, inc=1,
                            device_id=(self.recv_from,),
                            device_id_type=pl.DeviceIdType.MESH)
        return out

    def drain(self):
        self.sends[N_DEV - 3].wait_send()
        self.sends[N_DEV - 2].wait_send()
        pl.semaphore_wait(self.credit_sem, 2)


def kernel(x, w_mat):
    m_glob, _ = x.shape
    _, n = w_mat.shape
    m_per = m_glob // N_DEV
    n2 = n // 2

    def body(x_ref, w_ref, out_ref,
             sbuf_r, rbuf_r, sbuf_l, rbuf_l,
             ssems_r, rsems_r, ssems_l, rsems_l,
             credit_r, credit_l):
        my = lax.axis_index("i")
        left = lax.rem(my - 1 + N_DEV, N_DEV)
        right = lax.rem(my + 1, N_DEV)

        barrier = pltpu.get_barrier_semaphore()
        for nbr in (left, right):
            pl.semaphore_signal(barrier, inc=1, device_id=(nbr,),
                                device_id_type=pl.DeviceIdType.MESH)
        pl.semaphore_wait(barrier, 2)

        w = w_ref[:, :].astype(jnp.bfloat16)

        def chunk_partial(c, w_half):
            xs = x_ref[pl.ds(c * m_per, m_per), :].astype(jnp.bfloat16)
            return jnp.dot(xs, w_half, preferred_element_type=jnp.float32)

        ring_r = _Ring(send_to=right, recv_from=left, w_half=w[:, :n2],
                       send_buf=sbuf_r, recv_buf=rbuf_r,
                       send_sems=ssems_r, recv_sems=rsems_r,
                       credit_sem=credit_r)
        ring_l = _Ring(send_to=left, recv_from=right, w_half=w[:, n2:],
                       send_buf=sbuf_l, recv_buf=rbuf_l,
                       send_sems=ssems_l, recv_sems=rsems_l,
                       credit_sem=credit_l)

        for s in range(N_DEV - 1):
            c_r = lax.rem(my - 1 - s + 2 * N_DEV, N_DEV)
            c_l = lax.rem(my + 1 + s, N_DEV)
            part_r = chunk_partial(c_r, ring_r.w_half)
            part_l = chunk_partial(c_l, ring_l.w_half)
            ring_r.step(s, part_r)
            ring_l.step(s, part_l)

        part_r = chunk_partial(my, ring_r.w_half)
        part_l = chunk_partial(my, ring_l.w_half)
        out_ref[:, :n2] = ring_r.finish(part_r)
        out_ref[:, n2:] = ring_l.finish(part_l)
        ring_r.drain()
        ring_l.drain()

    return pl.pallas_call(
        body,
        out_shape=jax.ShapeDtypeStruct((m_per, n), jnp.float32),
        in_specs=[pl.BlockSpec(memory_space=pltpu.VMEM),
                  pl.BlockSpec(memory_space=pltpu.VMEM)],
        out_specs=pl.BlockSpec(memory_space=pltpu.VMEM),
        scratch_shapes=[
            pltpu.VMEM((2, m_per, n2), jnp.bfloat16),
            pltpu.VMEM((2, m_per, n2), jnp.bfloat16),
            pltpu.VMEM((2, m_per, n2), jnp.bfloat16),
            pltpu.VMEM((2, m_per, n2), jnp.bfloat16),
            pltpu.SemaphoreType.DMA((2,)),
            pltpu.SemaphoreType.DMA((2,)),
            pltpu.SemaphoreType.DMA((2,)),
            pltpu.SemaphoreType.DMA((2,)),
            pltpu.SemaphoreType.REGULAR,
            pltpu.SemaphoreType.REGULAR,
        ],
        compiler_params=pltpu.CompilerParams(collective_id=0),
    )(x, w_mat)


# device time: 439114 ns/iter; 1.8134x vs baseline; 1.6712x over previous
import jax
import jax.numpy as jnp
from jax import lax
from jax.experimental import pallas as pl
from jax.experimental.pallas import tpu as pltpu

N_DEV = 32


def _gelu(y):
    c = 0.7978845608028654
    return 0.5 * y * (1.0 + jnp.tanh(c * (y + 0.044715 * y * y * y)))


def _ring_order():
    import distributed_mesh_v7x as dm

    mesh = dm.get_mesh("i", world_size=N_DEV)
    coords = [tuple(d.coords) for d in mesh.devices.flat]
    pos_of = {c: i for i, c in enumerate(coords)}
    xs = sorted({c[0] for c in coords})
    ys = sorted({c[1] for c in coords})
    zs = sorted({c[2] for c in coords})
    if (len(xs) != 2 or len(zs) % 2
            or len(xs) * len(ys) * len(zs) != N_DEV
            or len(coords) != N_DEV):
        return list(range(N_DEV))
    ham = []
    for zi, z in enumerate(zs):
        for y in (ys if zi % 2 == 0 else ys[::-1]):
            ham.append((xs[0], y, z))
    for zi, z in enumerate(zs[::-1]):
        for y in (ys if zi % 2 == 0 else ys[::-1]):
            ham.append((xs[1], y, z))
    return [pos_of[c] for c in ham]


class _Ring:

    def __init__(self, *, send_to, recv_from, w_half, send_buf, recv_buf,
                 send_sems, recv_sems, credit_sem):
        self.send_to = send_to
        self.recv_from = recv_from
        self.w_half = w_half
        self.send_buf = send_buf
        self.recv_buf = recv_buf
        self.send_sems = send_sems
        self.recv_sems = recv_sems
        self.credit_sem = credit_sem
        self.sends = {}

    def recv_desc(self, slot):
        return pltpu.make_async_remote_copy(
            src_ref=self.send_buf.at[slot], dst_ref=self.recv_buf.at[slot],
            send_sem=self.send_sems.at[slot], recv_sem=self.recv_sems.at[slot],
            device_id=(self.recv_from,), device_id_type=pl.DeviceIdType.MESH)

    def step(self, s, part):
        slot = s % 2
        if s == 0:
            wire = part.astype(jnp.bfloat16)
        else:
            pslot = (s - 1) % 2
            self.recv_desc(pslot).wait_recv()
            wire = (part + self.recv_buf[pslot, :, :].astype(jnp.float32)
                    ).astype(jnp.bfloat16)
        if s >= 2:
            self.sends[s - 2].wait_send()
            pl.semaphore_wait(self.credit_sem, 1)
        self.send_buf[slot, :, :] = wire
        rdma = pltpu.make_async_remote_copy(
            src_ref=self.send_buf.at[slot], dst_ref=self.recv_buf.at[slot],
            send_sem=self.send_sems.at[slot], recv_sem=self.recv_sems.at[slot],
            device_id=(self.send_to,), device_id_type=pl.DeviceIdType.MESH)
        rdma.start()
        self.sends[s] = rdma
        if s >= 1:
            pl.semaphore_signal(self.credit_sem, inc=1,
                                device_id=(self.recv_from,),
                                device_id_type=pl.DeviceIdType.MESH)

    def finish(self, part):
        last_slot = (N_DEV - 2) % 2
        self.recv_desc(last_slot).wait_recv()
        out = _gelu(part + self.recv_buf[last_slot, :, :].astype(jnp.float32))
        pl.semaphore_signal(self.credit_sem, inc=1,
                            device_id=(self.recv_from,),
                            device_id_type=pl.DeviceIdType.MESH)
        return out

    def drain(self):
        self.sends[N_DEV - 3].wait_send()
        self.sends[N_DEV - 2].wait_send()
        pl.semaphore_wait(self.credit_sem, 2)


def kernel(x, w_mat):
    m_glob, _ = x.shape
    _, n = w_mat.shape
    m_per = m_glob // N_DEV
    n2 = n // 2

    sigma_list = _ring_order()
    inv_list = [0] * N_DEV
    for rr, pp in enumerate(sigma_list):
        inv_list[pp] = rr
    sigma = jnp.asarray(sigma_list, dtype=jnp.int32)
    inv = jnp.asarray(inv_list, dtype=jnp.int32)
    my = lax.axis_index("i")
    r = inv[my]
    hops = jnp.arange(N_DEV - 1, dtype=jnp.int32)
    meta = jnp.concatenate([
        jnp.stack([sigma[(r + 1) % N_DEV], sigma[(r - 1) % N_DEV]]),
        sigma[(r - 1 - hops) % N_DEV],
        sigma[(r + 1 + hops) % N_DEV],
    ]).astype(jnp.int32)

    def body(meta_ref, x_ref, w_ref, out_ref,
             sbuf_r, rbuf_r, sbuf_l, rbuf_l,
             ssems_r, rsems_r, ssems_l, rsems_l,
             credit_r, credit_l):
        nxt = meta_ref[0]
        prv = meta_ref[1]

        barrier = pltpu.get_barrier_semaphore()
        for nbr in (prv, nxt):
            pl.semaphore_signal(barrier, inc=1, device_id=(nbr,),
                                device_id_type=pl.DeviceIdType.MESH)
        pl.semaphore_wait(barrier, 2)

        w = w_ref[:, :].astype(jnp.bfloat16)

        def chunk_partial(c, w_half):
            xs = x_ref[pl.ds(c * m_per, m_per), :].astype(jnp.bfloat16)
            return jnp.dot(xs, w_half, preferred_element_type=jnp.float32)

        ring_r = _Ring(send_to=nxt, recv_from=prv, w_half=w[:, :n2],
                       send_buf=sbuf_r, recv_buf=rbuf_r,
                       send_sems=ssems_r, recv_sems=rsems_r,
                       credit_sem=credit_r)
        ring_l = _Ring(send_to=prv, recv_from=nxt, w_half=w[:, n2:],
                       send_buf=sbuf_l, recv_buf=rbuf_l,
                       send_sems=ssems_l, recv_sems=rsems_l,
                       credit_sem=credit_l)

        for s in range(N_DEV - 1):
            part_r = chunk_partial(meta_ref[2 + s], ring_r.w_half)
            part_l = chunk_partial(meta_ref[2 + (N_DEV - 1) + s], ring_l.w_half)
            ring_r.step(s, part_r)
            ring_l.step(s, part_l)

        mine = lax.axis_index("i")
        part_r = chunk_partial(mine, ring_r.w_half)
        part_l = chunk_partial(mine, ring_l.w_half)
        out_ref[:, :n2] = ring_r.finish(part_r)
        out_ref[:, n2:] = ring_l.finish(part_l)
        ring_r.drain()
        ring_l.drain()

    return pl.pallas_call(
        body,
        out_shape=jax.ShapeDtypeStruct((m_per, n), jnp.float32),
        in_specs=[pl.BlockSpec(memory_space=pltpu.SMEM),
                  pl.BlockSpec(memory_space=pltpu.VMEM),
                  pl.BlockSpec(memory_space=pltpu.VMEM)],
        out_specs=pl.BlockSpec(memory_space=pltpu.VMEM),
        scratch_shapes=[
            pltpu.VMEM((2, m_per, n2), jnp.bfloat16),
            pltpu.VMEM((2, m_per, n2), jnp.bfloat16),
            pltpu.VMEM((2, m_per, n2), jnp.bfloat16),
            pltpu.VMEM((2, m_per, n2), jnp.bfloat16),
            pltpu.SemaphoreType.DMA((2,)),
            pltpu.SemaphoreType.DMA((2,)),
            pltpu.SemaphoreType.DMA((2,)),
            pltpu.SemaphoreType.DMA((2,)),
            pltpu.SemaphoreType.REGULAR,
            pltpu.SemaphoreType.REGULAR,
        ],
        compiler_params=pltpu.CompilerParams(collective_id=0),
    )(meta, x, w_mat)


# device time: 369237 ns/iter; 2.1566x vs baseline; 1.1892x over previous
import jax
import jax.numpy as jnp
from jax import lax
from jax.experimental import pallas as pl
from jax.experimental.pallas import tpu as pltpu

N_DEV = 32
N_RING = 4


def _gelu(y):
    c = 0.7978845608028654
    return 0.5 * y * (1.0 + jnp.tanh(c * (y + 0.044715 * y * y * y)))


def _ring_order():
    import distributed_mesh_v7x as dm

    mesh = dm.get_mesh("i", world_size=N_DEV)
    coords = [tuple(d.coords) for d in mesh.devices.flat]
    pos_of = {c: i for i, c in enumerate(coords)}
    xs = sorted({c[0] for c in coords})
    ys = sorted({c[1] for c in coords})
    zs = sorted({c[2] for c in coords})
    if (len(xs) != 2 or len(zs) % 2
            or len(xs) * len(ys) * len(zs) != N_DEV
            or len(coords) != N_DEV):
        return list(range(N_DEV))
    ham = []
    for zi, z in enumerate(zs):
        for y in (ys if zi % 2 == 0 else ys[::-1]):
            ham.append((xs[0], y, z))
    for zi, z in enumerate(zs[::-1]):
        for y in (ys if zi % 2 == 0 else ys[::-1]):
            ham.append((xs[1], y, z))
    return [pos_of[c] for c in ham]


class _Ring:

    def __init__(self, *, send_to, recv_from, send_buf, recv_buf,
                 send_sems, recv_sems, credit_sem):
        self.send_to = send_to
        self.recv_from = recv_from
        self.send_buf = send_buf
        self.recv_buf = recv_buf
        self.send_sems = send_sems
        self.recv_sems = recv_sems
        self.credit_sem = credit_sem
        self.sends = {}

    def recv_desc(self, slot):
        return pltpu.make_async_remote_copy(
            src_ref=self.send_buf.at[slot], dst_ref=self.recv_buf.at[slot],
            send_sem=self.send_sems.at[slot], recv_sem=self.recv_sems.at[slot],
            device_id=(self.recv_from,), device_id_type=pl.DeviceIdType.MESH)

    def step(self, s, part):
        slot = s % 2
        if s == 0:
            wire = part.astype(jnp.bfloat16)
        else:
            pslot = (s - 1) % 2
            self.recv_desc(pslot).wait_recv()
            wire = (part + self.recv_buf[pslot, :, :].astype(jnp.float32)
                    ).astype(jnp.bfloat16)
        if s >= 2:
            self.sends[s - 2].wait_send()
            pl.semaphore_wait(self.credit_sem, 1)
        self.send_buf[slot, :, :] = wire
        rdma = pltpu.make_async_remote_copy(
            src_ref=self.send_buf.at[slot], dst_ref=self.recv_buf.at[slot],
            send_sem=self.send_sems.at[slot], recv_sem=self.recv_sems.at[slot],
            device_id=(self.send_to,), device_id_type=pl.DeviceIdType.MESH)
        rdma.start()
        self.sends[s] = rdma
        if s >= 1:
            pl.semaphore_signal(self.credit_sem, inc=1,
                                device_id=(self.recv_from,),
                                device_id_type=pl.DeviceIdType.MESH)

    def finish(self, part):
        last_slot = (N_DEV - 2) % 2
        self.recv_desc(last_slot).wait_recv()
        out = _gelu(part + self.recv_buf[last_slot, :, :].astype(jnp.float32))
        pl.semaphore_signal(self.credit_sem, inc=1,
                            device_id=(self.recv_from,),
                            device_id_type=pl.DeviceIdType.MESH)
        return out

    def drain(self):
        self.sends[N_DEV - 3].wait_send()
        self.sends[N_DEV - 2].wait_send()
        pl.semaphore_wait(self.credit_sem, 2)


def kernel(x, w_mat):
    m_glob, _ = x.shape
    _, n = w_mat.shape
    m_per = m_glob // N_DEV
    n2 = n // 2
    n4 = n // N_RING

    sigma_list = _ring_order()
    inv_list = [0] * N_DEV
    for rr, pp in enumerate(sigma_list):
        inv_list[pp] = rr
    sigma = jnp.asarray(sigma_list, dtype=jnp.int32)
    inv = jnp.asarray(inv_list, dtype=jnp.int32)
    my = lax.axis_index("i")
    r = inv[my]
    hops = jnp.arange(N_DEV - 1, dtype=jnp.int32)
    meta = jnp.concatenate([
        jnp.stack([sigma[(r + 1) % N_DEV], sigma[(r - 1) % N_DEV]]),
        sigma[(r - 1 - hops) % N_DEV],
        sigma[(r + 1 + hops) % N_DEV],
    ]).astype(jnp.int32)

    def body(meta_ref, x_ref, w_ref, out_ref, *scratch):
        sbufs = scratch[0:N_RING]
        rbufs = scratch[N_RING:2 * N_RING]
        ssems = scratch[2 * N_RING:3 * N_RING]
        rsems = scratch[3 * N_RING:4 * N_RING]
        credits = scratch[4 * N_RING:5 * N_RING]

        nxt = meta_ref[0]
        prv = meta_ref[1]

        barrier = pltpu.get_barrier_semaphore()
        for nbr in (prv, nxt):
            pl.semaphore_signal(barrier, inc=1, device_id=(nbr,),
                                device_id_type=pl.DeviceIdType.MESH)
        pl.semaphore_wait(barrier, 2)

        w = w_ref[:, :].astype(jnp.bfloat16)

        def chunk_partial(c, w_half):
            xs = x_ref[pl.ds(c * m_per, m_per), :].astype(jnp.bfloat16)
            return jnp.dot(xs, w_half, preferred_element_type=jnp.float32)

        rings = []
        for i in range(N_RING):
            fwd = i < N_RING // 2
            rings.append(_Ring(
                send_to=nxt if fwd else prv,
                recv_from=prv if fwd else nxt,
                send_buf=sbufs[i], recv_buf=rbufs[i],
                send_sems=ssems[i], recv_sems=rsems[i],
                credit_sem=credits[i]))

        def parts_at(c_r, c_l):
            pr = chunk_partial(c_r, w[:, :n2])
            plf = chunk_partial(c_l, w[:, n2:])
            return [pr[:, :n4], pr[:, n4:], plf[:, :n4], plf[:, n4:]]

        order = (0, 2, 1, 3)

        for s in range(N_DEV - 1):
            parts = parts_at(meta_ref[2 + s], meta_ref[2 + (N_DEV - 1) + s])
            for i in order:
                rings[i].step(s, parts[i])

        mine = lax.axis_index("i")
        parts = parts_at(mine, mine)
        for i in order:
            out_ref[:, i * n4:(i + 1) * n4] = rings[i].finish(parts[i])
        for i in order:
            rings[i].drain()

    return pl.pallas_call(
        body,
        out_shape=jax.ShapeDtypeStruct((m_per, n), jnp.float32),
        in_specs=[pl.BlockSpec(memory_space=pltpu.SMEM),
                  pl.BlockSpec(memory_space=pltpu.VMEM),
                  pl.BlockSpec(memory_space=pltpu.VMEM)],
        out_specs=pl.BlockSpec(memory_space=pltpu.VMEM),
        scratch_shapes=(
            [pltpu.VMEM((2, m_per, n4), jnp.bfloat16)] * N_RING
            + [pltpu.VMEM((2, m_per, n4), jnp.bfloat16)] * N_RING
            + [pltpu.SemaphoreType.DMA((2,))] * N_RING
            + [pltpu.SemaphoreType.DMA((2,))] * N_RING
            + [pltpu.SemaphoreType.REGULAR] * N_RING
        ),
        compiler_params=pltpu.CompilerParams(collective_id=0),
    )(meta, x, w_mat)


# device time: 368965 ns/iter; 2.1581x vs baseline; 1.0007x over previous
import jax
import jax.numpy as jnp
from jax import lax
from jax.experimental import pallas as pl
from jax.experimental.pallas import tpu as pltpu

N_DEV = 32
N_RING = 8


def _gelu(y):
    c = 0.7978845608028654
    return 0.5 * y * (1.0 + jnp.tanh(c * (y + 0.044715 * y * y * y)))


def _ring_order():
    import distributed_mesh_v7x as dm

    mesh = dm.get_mesh("i", world_size=N_DEV)
    coords = [tuple(d.coords) for d in mesh.devices.flat]
    pos_of = {c: i for i, c in enumerate(coords)}
    xs = sorted({c[0] for c in coords})
    ys = sorted({c[1] for c in coords})
    zs = sorted({c[2] for c in coords})
    if (len(xs) != 2 or len(zs) % 2
            or len(xs) * len(ys) * len(zs) != N_DEV
            or len(coords) != N_DEV):
        return list(range(N_DEV))
    ham = []
    for zi, z in enumerate(zs):
        for y in (ys if zi % 2 == 0 else ys[::-1]):
            ham.append((xs[0], y, z))
    for zi, z in enumerate(zs[::-1]):
        for y in (ys if zi % 2 == 0 else ys[::-1]):
            ham.append((xs[1], y, z))
    return [pos_of[c] for c in ham]


class _Ring:

    def __init__(self, *, send_to, recv_from, send_buf, recv_buf,
                 send_sems, recv_sems, credit_sem):
        self.send_to = send_to
        self.recv_from = recv_from
        self.send_buf = send_buf
        self.recv_buf = recv_buf
        self.send_sems = send_sems
        self.recv_sems = recv_sems
        self.credit_sem = credit_sem
        self.sends = {}

    def recv_desc(self, slot):
        return pltpu.make_async_remote_copy(
            src_ref=self.send_buf.at[slot], dst_ref=self.recv_buf.at[slot],
            send_sem=self.send_sems.at[slot], recv_sem=self.recv_sems.at[slot],
            device_id=(self.recv_from,), device_id_type=pl.DeviceIdType.MESH)

    def step(self, s, part):
        slot = s % 2
        if s == 0:
            wire = part.astype(jnp.bfloat16)
        else:
            pslot = (s - 1) % 2
            self.recv_desc(pslot).wait_recv()
            wire = (part + self.recv_buf[pslot, :, :].astype(jnp.float32)
                    ).astype(jnp.bfloat16)
        if s >= 2:
            self.sends[s - 2].wait_send()
            pl.semaphore_wait(self.credit_sem, 1)
        self.send_buf[slot, :, :] = wire
        rdma = pltpu.make_async_remote_copy(
            src_ref=self.send_buf.at[slot], dst_ref=self.recv_buf.at[slot],
            send_sem=self.send_sems.at[slot], recv_sem=self.recv_sems.at[slot],
            device_id=(self.send_to,), device_id_type=pl.DeviceIdType.MESH)
        rdma.start()
        self.sends[s] = rdma
        if s >= 1:
            pl.semaphore_signal(self.credit_sem, inc=1,
                                device_id=(self.recv_from,),
                                device_id_type=pl.DeviceIdType.MESH)

    def finish(self, part):
        last_slot = (N_DEV - 2) % 2
        self.recv_desc(last_slot).wait_recv()
        out = _gelu(part + self.recv_buf[last_slot, :, :].astype(jnp.float32))
        pl.semaphore_signal(self.credit_sem, inc=1,
                            device_id=(self.recv_from,),
                            device_id_type=pl.DeviceIdType.MESH)
        return out

    def drain(self):
        self.sends[N_DEV - 3].wait_send()
        self.sends[N_DEV - 2].wait_send()
        pl.semaphore_wait(self.credit_sem, 2)


def kernel(x, w_mat):
    m_glob, _ = x.shape
    _, n = w_mat.shape
    m_per = m_glob // N_DEV
    n2 = n // 2
    n4 = n // N_RING

    sigma_list = _ring_order()
    inv_list = [0] * N_DEV
    for rr, pp in enumerate(sigma_list):
        inv_list[pp] = rr
    sigma = jnp.asarray(sigma_list, dtype=jnp.int32)
    inv = jnp.asarray(inv_list, dtype=jnp.int32)
    my = lax.axis_index("i")
    r = inv[my]
    hops = jnp.arange(N_DEV - 1, dtype=jnp.int32)
    meta = jnp.concatenate([
        jnp.stack([sigma[(r + 1) % N_DEV], sigma[(r - 1) % N_DEV]]),
        sigma[(r - 1 - hops) % N_DEV],
        sigma[(r + 1 + hops) % N_DEV],
    ]).astype(jnp.int32)

    def body(meta_ref, x_ref, w_ref, out_ref, *scratch):
        sbufs = scratch[0:N_RING]
        rbufs = scratch[N_RING:2 * N_RING]
        ssems = scratch[2 * N_RING:3 * N_RING]
        rsems = scratch[3 * N_RING:4 * N_RING]
        credits = scratch[4 * N_RING:5 * N_RING]

        nxt = meta_ref[0]
        prv = meta_ref[1]

        barrier = pltpu.get_barrier_semaphore()
        for nbr in (prv, nxt):
            pl.semaphore_signal(barrier, inc=1, device_id=(nbr,),
                                device_id_type=pl.DeviceIdType.MESH)
        pl.semaphore_wait(barrier, 2)

        w = w_ref[:, :].astype(jnp.bfloat16)

        def chunk_partial(c, w_half):
            xs = x_ref[pl.ds(c * m_per, m_per), :].astype(jnp.bfloat16)
            return jnp.dot(xs, w_half, preferred_element_type=jnp.float32)

        rings = []
        for i in range(N_RING):
            fwd = i < N_RING // 2
            rings.append(_Ring(
                send_to=nxt if fwd else prv,
                recv_from=prv if fwd else nxt,
                send_buf=sbufs[i], recv_buf=rbufs[i],
                send_sems=ssems[i], recv_sems=rsems[i],
                credit_sem=credits[i]))

        half = N_RING // 2

        def parts_at(c_r, c_l):
            pr = chunk_partial(c_r, w[:, :n2])
            plf = chunk_partial(c_l, w[:, n2:])
            return ([pr[:, j * n4:(j + 1) * n4] for j in range(half)]
                    + [plf[:, j * n4:(j + 1) * n4] for j in range(half)])

        order = [i for pair in zip(range(half), range(half, N_RING))
                 for i in pair]

        for s in range(N_DEV - 1):
            parts = parts_at(meta_ref[2 + s], meta_ref[2 + (N_DEV - 1) + s])
            for i in order:
                rings[i].step(s, parts[i])

        mine = lax.axis_index("i")
        parts = parts_at(mine, mine)
        for i in order:
            out_ref[:, i * n4:(i + 1) * n4] = rings[i].finish(parts[i])
        for i in order:
            rings[i].drain()

    return pl.pallas_call(
        body,
        out_shape=jax.ShapeDtypeStruct((m_per, n), jnp.float32),
        in_specs=[pl.BlockSpec(memory_space=pltpu.SMEM),
                  pl.BlockSpec(memory_space=pltpu.VMEM),
                  pl.BlockSpec(memory_space=pltpu.VMEM)],
        out_specs=pl.BlockSpec(memory_space=pltpu.VMEM),
        scratch_shapes=(
            [pltpu.VMEM((2, m_per, n4), jnp.bfloat16)] * N_RING
            + [pltpu.VMEM((2, m_per, n4), jnp.bfloat16)] * N_RING
            + [pltpu.SemaphoreType.DMA((2,))] * N_RING
            + [pltpu.SemaphoreType.DMA((2,))] * N_RING
            + [pltpu.SemaphoreType.REGULAR] * N_RING
        ),
        compiler_params=pltpu.CompilerParams(collective_id=0),
    )(meta, x, w_mat)
